# baseline (speedup 1.0000x reference)
import functools
import math
import statistics as _stats

import jax
import jax.numpy as jnp
from jax import lax
from jax.experimental import pallas as pl
from jax.experimental.pallas import tpu as pltpu
from jax.experimental.pallas import tpu_sc as plsc

_EPS = 1e-12
_SEL_CAP = 64
_TARGET_RANK = 2048


def _pick_block(K: int) -> int:
    for blk in (2000, 1600, 1280, 1024, 1000, 800, 640, 512, 500, 400, 320,
                256, 250, 200, 160, 128, 125, 104, 100, 80, 64, 56, 50, 40,
                32, 25, 24, 20, 16, 10, 8):
        if K % blk == 0 and blk % 8 == 0:
            return blk
    return K


def _sc_gather_rows(keys, idx):
    K, D = keys.shape
    B = idx.shape[0]
    info = plsc.get_sparse_core_info()
    nw = info.num_cores * info.num_subcores
    b_per_w = B // nw
    mesh = plsc.VectorSubcoreMesh(core_axis_name="c", subcore_axis_name="s")

    @functools.partial(
        pl.kernel,
        mesh=mesh,
        out_type=jax.ShapeDtypeStruct((B, D), jnp.float32),
        scratch_types=[
            pltpu.VMEM((b_per_w,), jnp.int32),
            pltpu.VMEM((b_per_w, D), jnp.float32),
            pltpu.SemaphoreType.DMA,
        ],
    )
    def gather_kernel(keys_hbm, idx_hbm, out_hbm, idx_v, rows_v, sem):
        wid = lax.axis_index("s") * info.num_cores + lax.axis_index("c")
        base = wid * b_per_w
        pltpu.sync_copy(idx_hbm.at[pl.ds(base, b_per_w)], idx_v)
        pltpu.async_copy(keys_hbm.at[idx_v], rows_v, sem).wait()
        pltpu.sync_copy(rows_v, out_hbm.at[pl.ds(base, b_per_w)])

    return gather_kernel(keys, idx)


def _gtsim_body(yhat_ref, gtrows_ref, out_ref):
    y = yhat_ref[...]
    qn = y / (jnp.sqrt(jnp.sum(y * y, axis=1, keepdims=True)) + _EPS)
    g = gtrows_ref[...]
    gn = g / (jnp.sqrt(jnp.sum(g * g, axis=1, keepdims=True)) + _EPS)
    out_ref[...] = jnp.sum(qn * gn, axis=1, keepdims=True)


def _tc_gtsim(y_hat, gt_rows):
    B, D = y_hat.shape
    return pl.pallas_call(
        _gtsim_body,
        out_shape=jax.ShapeDtypeStruct((B, 1), jnp.float32),
    )(y_hat, gt_rows)


def _sc_select(gtsim, y_hat, gt_idx, thresh):
    B, D = y_hat.shape
    S = _SEL_CAP
    L = 16
    n_chunks = B // L
    s_chunks = S // L
    rows_per_tile = 8
    n_gather_tiles = S // rows_per_tile
    mesh = plsc.VectorSubcoreMesh(
        core_axis_name="c", subcore_axis_name="s", num_cores=1)

    @functools.partial(
        pl.kernel,
        mesh=mesh,
        compiler_params=pltpu.CompilerParams(needs_layout_passes=False),
        out_type=(
            jax.ShapeDtypeStruct((S, D), jnp.float32),
            jax.ShapeDtypeStruct((S,), jnp.float32),
            jax.ShapeDtypeStruct((S,), jnp.int32),
            jax.ShapeDtypeStruct((S,), jnp.int32),
            jax.ShapeDtypeStruct((B,), jnp.int32),
        ),
        scratch_types=[
            pltpu.VMEM((B,), jnp.float32),
            pltpu.VMEM((B,), jnp.int32),
            pltpu.VMEM((B,), jnp.int32),
            pltpu.VMEM((S,), jnp.int32),
            pltpu.VMEM((S,), jnp.float32),
            pltpu.VMEM((S,), jnp.int32),
            pltpu.VMEM((S,), jnp.int32),
            pltpu.VMEM((rows_per_tile,), jnp.int32),
            pltpu.VMEM((rows_per_tile, D), jnp.float32),
            pltpu.VMEM_SHARED((S,), jnp.int32),
            pltpu.SemaphoreType.DMA,
        ],
    )
    def select_kernel(gtsim_hbm, yhat_hbm, gtidx_hbm,
                      qsel_hbm, selg_hbm, seli_hbm, valid_hbm, flag_hbm,
                      gtsim_v, gtidx_v, flag_v, selidx_v, selg_v, seli_v,
                      valid_v, idx_v, rows_v, shared_idx, sem):
        w = lax.axis_index("s")

        @pl.when(w == 0)
        def _compact():
            pltpu.sync_copy(gtsim_hbm, gtsim_v)
            pltpu.sync_copy(gtidx_hbm, gtidx_v)
            for c in range(s_chunks):
                selidx_v[pl.ds(c * L, L)] = jnp.zeros((L,), jnp.int32)
                valid_v[pl.ds(c * L, L)] = jnp.zeros((L,), jnp.int32)

            ones = jnp.ones((L,), jnp.int32)
            off_v = jnp.zeros((L,), jnp.int32)
            thresh_v = jnp.full((L,), thresh, jnp.float32)
            cap_v = jnp.full((L,), S, jnp.int32)
            lane = lax.iota(jnp.int32, L)
            for c in range(n_chunks):
                v = gtsim_v[pl.ds(c * L, L)]
                m = v > thresh_v
                mi = m.astype(jnp.int32)
                cs = plsc.cumsum(mi)
                pos = cs - mi + off_v
                okm = m & (pos < cap_v)
                ids = jnp.full((L,), c * L, jnp.int32) + lane
                plsc.store_scatter(selidx_v, [pos], ids, mask=okm)
                plsc.store_scatter(valid_v, [pos], ones, mask=okm)
                flag_v[pl.ds(c * L, L)] = okm.astype(jnp.int32)
                off_v = off_v + plsc.cummax(lax.rev(cs, (0,)))
            for c in range(s_chunks):
                idxv = selidx_v[pl.ds(c * L, L)]
                selg_v[pl.ds(c * L, L)] = plsc.load_gather(gtsim_v, [idxv])
                seli_v[pl.ds(c * L, L)] = plsc.load_gather(gtidx_v, [idxv])
            pltpu.sync_copy(selg_v, selg_hbm)
            pltpu.sync_copy(seli_v, seli_hbm)
            pltpu.sync_copy(valid_v, valid_hbm)
            pltpu.sync_copy(flag_v, flag_hbm)
            pltpu.sync_copy(selidx_v, shared_idx)

        plsc.subcore_barrier()

        @pl.when(w < n_gather_tiles)
        def _gather_rows():
            base = w * rows_per_tile
            pltpu.sync_copy(shared_idx.at[pl.ds(base, rows_per_tile)], idx_v)
            pltpu.async_copy(yhat_hbm.at[idx_v], rows_v, sem).wait()
            pltpu.sync_copy(rows_v, qsel_hbm.at[pl.ds(base, rows_per_tile)])

    return select_kernel(gtsim, y_hat, gt_idx)


def _count_body(nsteps, blk, B, D, K, S,
                gtsim_ref, flag_ref, yhat_ref, keys_ref,
                qselraw_ref, selg_ref, seli_ref, valid_ref,
                out_ref, qn16_ref, qselh_ref, qsell_ref, cnt_ref, cnts_ref,
                kh_a, kl_a, kh_b, kl_b):
    step = pl.program_id(0)

    @pl.when(step == 0)
    def _prologue():
        y = yhat_ref[...]
        qn = y / (jnp.sqrt(jnp.sum(y * y, axis=1, keepdims=True)) + _EPS)
        qn16_ref[...] = qn.astype(jnp.bfloat16)
        qs = qselraw_ref[...]
        qsel = qs / (jnp.sqrt(jnp.sum(qs * qs, axis=1,
                                      keepdims=True)) + _EPS)
        qh = qsel.astype(jnp.bfloat16)
        qselh_ref[...] = qh
        qsell_ref[...] = (qsel - qh.astype(jnp.float32)).astype(jnp.bfloat16)
        cnt_ref[...] = jnp.zeros_like(cnt_ref)
        cnts_ref[...] = jnp.zeros_like(cnts_ref)

    def _normalize_into(kh_ref, kl_ref):
        kb = keys_ref[...]
        inv = 1.0 / (jnp.sqrt(jnp.sum(kb * kb, axis=1, keepdims=True))
                     + _EPS)
        kn = kb * inv
        kn16 = kn.astype(jnp.bfloat16)
        kh_ref[...] = kn16
        kl_ref[...] = (kn - kn16.astype(jnp.float32)).astype(jnp.bfloat16)

    def _count_from(kh_ref, kl_ref):
        kn16 = kh_ref[...]
        sim = lax.dot_general(qn16_ref[...], kn16, (((1,), (1,)), ((), ())),
                              preferred_element_type=jnp.float32,
                              precision=lax.Precision.DEFAULT)
        cnt_ref[...] += jnp.sum((sim > gtsim_ref[...]).astype(jnp.int32),
                                axis=1, keepdims=True)

        knl = kl_ref[...]
        dnum = (((1,), (1,)), ((), ()))

        def _dot(a, b):
            return lax.dot_general(a, b, dnum,
                                   preferred_element_type=jnp.float32,
                                   precision=lax.Precision.DEFAULT)

        sims = (_dot(kn16, qselh_ref[...]) + _dot(knl, qselh_ref[...])
                + _dot(kn16, qsell_ref[...]))
        rows = ((step - 1) * blk
                + lax.broadcasted_iota(jnp.int32, (blk, S), 0))
        hits = (sims > selg_ref[...]) & (rows != seli_ref[...])
        if K % blk != 0:
            hits = hits & (rows < K)
        cnts_ref[...] += jnp.sum(hits.astype(jnp.int32), axis=0,
                                 keepdims=True)

    parity = lax.rem(step, 2)

    @pl.when((step > 0) & (parity == 1))
    def _count_a():
        _count_from(kh_a, kl_a)

    @pl.when((step > 0) & (parity == 0))
    def _count_b():
        _count_from(kh_b, kl_b)

    @pl.when((step < nsteps) & (parity == 0))
    def _fill_a():
        _normalize_into(kh_a, kl_a)

    @pl.when((step < nsteps) & (parity == 1))
    def _fill_b():
        _normalize_into(kh_b, kl_b)

    @pl.when(step == nsteps)
    def _epilogue():
        rank_b = (cnt_ref[...] + 1).astype(jnp.float32)
        keep = (1 - flag_ref[...]).astype(jnp.float32)
        s_fast = jnp.sum(keep / rank_b)
        rank_s = (cnts_ref[...] + 1).astype(jnp.float32)
        s_exact = jnp.sum(valid_ref[...].astype(jnp.float32) / rank_s)
        out_ref[...] = ((s_fast + s_exact) / B).reshape(1, 1)


def kernel(y_hat, keys, gt_idx):
    B, D = y_hat.shape
    K = keys.shape[0]
    S = _SEL_CAP
    gt_idx = gt_idx.astype(jnp.int32)

    gt_rows = _sc_gather_rows(keys, gt_idx)
    gtsim = _tc_gtsim(y_hat, gt_rows)

    frac = min(max(_TARGET_RANK / K, 1e-6), 0.5)
    thresh = _stats.NormalDist().inv_cdf(1.0 - frac) / math.sqrt(D)

    qsel_raw, sel_gtsim, sel_gtidx, sel_valid, sel_flag = _sc_select(
        gtsim.reshape(B), y_hat, gt_idx, thresh)

    blk = _pick_block(K)
    nsteps = pl.cdiv(K, blk)
    body = functools.partial(_count_body, nsteps, blk, B, D, K, S)
    out = pl.pallas_call(
        body,
        grid=(nsteps + 1,),
        in_specs=[
            pl.BlockSpec((B, 1), lambda k: (0, 0)),
            pl.BlockSpec((B, 1), lambda k: (0, 0)),
            pl.BlockSpec((B, D), lambda k: (0, 0)),
            pl.BlockSpec((blk, D),
                         lambda k: (jnp.minimum(k, nsteps - 1), 0)),
            pl.BlockSpec((S, D), lambda k: (0, 0)),
            pl.BlockSpec((1, S), lambda k: (0, 0)),
            pl.BlockSpec((1, S), lambda k: (0, 0)),
            pl.BlockSpec((1, S), lambda k: (0, 0)),
        ],
        out_specs=pl.BlockSpec((1, 1), lambda k: (0, 0)),
        out_shape=jax.ShapeDtypeStruct((1, 1), jnp.float32),
        scratch_shapes=[
            pltpu.VMEM((B, D), jnp.bfloat16),
            pltpu.VMEM((S, D), jnp.bfloat16),
            pltpu.VMEM((S, D), jnp.bfloat16),
            pltpu.VMEM((B, 1), jnp.int32),
            pltpu.VMEM((1, S), jnp.int32),
            pltpu.VMEM((blk, D), jnp.bfloat16),
            pltpu.VMEM((blk, D), jnp.bfloat16),
            pltpu.VMEM((blk, D), jnp.bfloat16),
            pltpu.VMEM((blk, D), jnp.bfloat16),
        ],
    )(gtsim, sel_flag.reshape(B, 1), y_hat, keys,
      qsel_raw, sel_gtsim.reshape(1, S), sel_gtidx.reshape(1, S),
      sel_valid.reshape(1, S))
    return out.reshape(())


# speedup vs baseline: 1.1794x; 1.1794x over previous
import functools
import math
import statistics as _stats

import jax
import jax.numpy as jnp
from jax import lax
from jax.experimental import pallas as pl
from jax.experimental.pallas import tpu as pltpu
from jax.experimental.pallas import tpu_sc as plsc

_EPS = 1e-12
_SEL_CAP = 64
_TARGET_RANK = 2048


def _pick_block(K: int) -> int:
    for blk in (4000, 1600, 1280, 1024, 2000, 800, 640, 512, 500, 400, 320,
                256, 250, 200, 160, 128, 125, 104, 100, 80, 64, 56, 50, 40,
                32, 25, 24, 20, 16, 10, 8):
        if K % blk == 0 and blk % 8 == 0:
            return blk
    return K


def _sc_gather_rows(keys, idx):
    K, D = keys.shape
    B = idx.shape[0]
    info = plsc.get_sparse_core_info()
    nw = info.num_cores * info.num_subcores
    b_per_w = B // nw
    mesh = plsc.VectorSubcoreMesh(core_axis_name="c", subcore_axis_name="s")

    @functools.partial(
        pl.kernel,
        mesh=mesh,
        out_type=jax.ShapeDtypeStruct((B, D), jnp.float32),
        scratch_types=[
            pltpu.VMEM((b_per_w,), jnp.int32),
            pltpu.VMEM((b_per_w, D), jnp.float32),
            pltpu.SemaphoreType.DMA,
        ],
    )
    def gather_kernel(keys_hbm, idx_hbm, out_hbm, idx_v, rows_v, sem):
        wid = lax.axis_index("s") * info.num_cores + lax.axis_index("c")
        base = wid * b_per_w
        pltpu.sync_copy(idx_hbm.at[pl.ds(base, b_per_w)], idx_v)
        pltpu.async_copy(keys_hbm.at[idx_v], rows_v, sem).wait()
        pltpu.sync_copy(rows_v, out_hbm.at[pl.ds(base, b_per_w)])

    return gather_kernel(keys, idx)


def _prep_body(yhat_ref, gtrows_ref, gtsim_ref, qn_ref, qn16_ref):
    y = yhat_ref[...]
    qn = y / (jnp.sqrt(jnp.sum(y * y, axis=1, keepdims=True)) + _EPS)
    qn_ref[...] = qn
    qn16_ref[...] = qn.astype(jnp.bfloat16)
    g = gtrows_ref[...]
    gn = g / (jnp.sqrt(jnp.sum(g * g, axis=1, keepdims=True)) + _EPS)
    gtsim_ref[...] = jnp.sum(qn * gn, axis=1, keepdims=True)


def _tc_prep(y_hat, gt_rows):
    B, D = y_hat.shape
    return pl.pallas_call(
        _prep_body,
        out_shape=(
            jax.ShapeDtypeStruct((B, 1), jnp.float32),
            jax.ShapeDtypeStruct((B, D), jnp.float32),
            jax.ShapeDtypeStruct((B, D), jnp.bfloat16),
        ),
    )(y_hat, gt_rows)


def _sc_select(gtsim, y_hat, gt_idx, thresh):
    B, D = y_hat.shape
    S = _SEL_CAP
    L = 16
    n_chunks = B // L
    s_chunks = S // L
    rows_per_tile = 8
    n_gather_tiles = S // rows_per_tile
    mesh = plsc.VectorSubcoreMesh(
        core_axis_name="c", subcore_axis_name="s", num_cores=1)

    @functools.partial(
        pl.kernel,
        mesh=mesh,
        compiler_params=pltpu.CompilerParams(needs_layout_passes=False),
        out_type=(
            jax.ShapeDtypeStruct((S, D), jnp.float32),
            jax.ShapeDtypeStruct((S,), jnp.float32),
            jax.ShapeDtypeStruct((S,), jnp.int32),
            jax.ShapeDtypeStruct((S,), jnp.int32),
            jax.ShapeDtypeStruct((B,), jnp.int32),
        ),
        scratch_types=[
            pltpu.VMEM((B,), jnp.float32),
            pltpu.VMEM((B,), jnp.int32),
            pltpu.VMEM((B,), jnp.int32),
            pltpu.VMEM((S,), jnp.int32),
            pltpu.VMEM((S,), jnp.float32),
            pltpu.VMEM((S,), jnp.int32),
            pltpu.VMEM((S,), jnp.int32),
            pltpu.VMEM((rows_per_tile,), jnp.int32),
            pltpu.VMEM((rows_per_tile, D), jnp.float32),
            pltpu.VMEM_SHARED((S,), jnp.int32),
            pltpu.SemaphoreType.DMA,
        ],
    )
    def select_kernel(gtsim_hbm, yhat_hbm, gtidx_hbm,
                      qsel_hbm, selg_hbm, seli_hbm, valid_hbm, flag_hbm,
                      gtsim_v, gtidx_v, flag_v, selidx_v, selg_v, seli_v,
                      valid_v, idx_v, rows_v, shared_idx, sem):
        w = lax.axis_index("s")

        @pl.when(w == 0)
        def _compact():
            pltpu.sync_copy(gtsim_hbm, gtsim_v)
            pltpu.sync_copy(gtidx_hbm, gtidx_v)
            for c in range(s_chunks):
                selidx_v[pl.ds(c * L, L)] = jnp.zeros((L,), jnp.int32)
                valid_v[pl.ds(c * L, L)] = jnp.zeros((L,), jnp.int32)

            ones = jnp.ones((L,), jnp.int32)
            off_v = jnp.zeros((L,), jnp.int32)
            thresh_v = jnp.full((L,), thresh, jnp.float32)
            cap_v = jnp.full((L,), S, jnp.int32)
            lane = lax.iota(jnp.int32, L)
            for c in range(n_chunks):
                v = gtsim_v[pl.ds(c * L, L)]
                m = v > thresh_v
                mi = m.astype(jnp.int32)
                cs = plsc.cumsum(mi)
                pos = cs - mi + off_v
                okm = m & (pos < cap_v)
                ids = jnp.full((L,), c * L, jnp.int32) + lane
                plsc.store_scatter(selidx_v, [pos], ids, mask=okm)
                plsc.store_scatter(valid_v, [pos], ones, mask=okm)
                flag_v[pl.ds(c * L, L)] = okm.astype(jnp.int32)
                off_v = off_v + plsc.cummax(lax.rev(cs, (0,)))
            for c in range(s_chunks):
                idxv = selidx_v[pl.ds(c * L, L)]
                selg_v[pl.ds(c * L, L)] = plsc.load_gather(gtsim_v, [idxv])
                seli_v[pl.ds(c * L, L)] = plsc.load_gather(gtidx_v, [idxv])
            pltpu.sync_copy(selg_v, selg_hbm)
            pltpu.sync_copy(seli_v, seli_hbm)
            pltpu.sync_copy(valid_v, valid_hbm)
            pltpu.sync_copy(flag_v, flag_hbm)
            pltpu.sync_copy(selidx_v, shared_idx)

        plsc.subcore_barrier()

        @pl.when(w < n_gather_tiles)
        def _gather_rows():
            base = w * rows_per_tile
            pltpu.sync_copy(shared_idx.at[pl.ds(base, rows_per_tile)], idx_v)
            pltpu.async_copy(yhat_hbm.at[idx_v], rows_v, sem).wait()
            pltpu.sync_copy(rows_v, qsel_hbm.at[pl.ds(base, rows_per_tile)])

    return select_kernel(gtsim, y_hat, gt_idx)


def _count_body(nsteps, blk, B, D, K, S,
                gtsim_ref, flag_ref, qn16_ref, keys_ref,
                qsel_in_ref, selg_ref, seli_ref, valid_ref,
                out_ref, qselh_ref, qsell_ref, cnt_ref, cnts_ref):
    step = pl.program_id(0)

    @pl.when(step == 0)
    def _prologue():
        qsel = qsel_in_ref[...]
        qh = qsel.astype(jnp.bfloat16)
        qselh_ref[...] = qh
        qsell_ref[...] = (qsel - qh.astype(jnp.float32)).astype(jnp.bfloat16)
        cnt_ref[...] = jnp.zeros_like(cnt_ref)
        cnts_ref[...] = jnp.zeros_like(cnts_ref)

    kb = keys_ref[...]
    inv = 1.0 / (jnp.sqrt(jnp.sum(kb * kb, axis=1, keepdims=True)) + _EPS)
    kn = kb * inv
    kn16 = kn.astype(jnp.bfloat16)

    sim = lax.dot_general(qn16_ref[...], kn16, (((1,), (1,)), ((), ())),
                          preferred_element_type=jnp.float32,
                          precision=lax.Precision.DEFAULT)
    cnt_ref[...] += jnp.sum((sim > gtsim_ref[...]).astype(jnp.int32),
                            axis=1, keepdims=True)

    knl = (kn - kn16.astype(jnp.float32)).astype(jnp.bfloat16)
    dnum = (((1,), (1,)), ((), ()))

    def _dot(a, b):
        return lax.dot_general(a, b, dnum,
                               preferred_element_type=jnp.float32,
                               precision=lax.Precision.DEFAULT)

    sims = (_dot(kn16, qselh_ref[...]) + _dot(knl, qselh_ref[...])
            + _dot(kn16, qsell_ref[...]))
    rows = step * blk + lax.broadcasted_iota(jnp.int32, (blk, S), 0)
    hits = (sims > selg_ref[...]) & (rows != seli_ref[...])
    if K % blk != 0:
        hits = hits & (rows < K)
    cnts_ref[...] += jnp.sum(hits.astype(jnp.int32), axis=0, keepdims=True)

    @pl.when(step == nsteps - 1)
    def _epilogue():
        rank_b = (cnt_ref[...] + 1).astype(jnp.float32)
        keep = (1 - flag_ref[...]).astype(jnp.float32)
        s_fast = jnp.sum(keep / rank_b)
        rank_s = (cnts_ref[...] + 1).astype(jnp.float32)
        s_exact = jnp.sum(valid_ref[...].astype(jnp.float32) / rank_s)
        out_ref[...] = ((s_fast + s_exact) / B).reshape(1, 1)


def kernel(y_hat, keys, gt_idx):
    B, D = y_hat.shape
    K = keys.shape[0]
    S = _SEL_CAP
    gt_idx = gt_idx.astype(jnp.int32)

    gt_rows = _sc_gather_rows(keys, gt_idx)
    gtsim, qn, qn16 = _tc_prep(y_hat, gt_rows)

    frac = min(max(_TARGET_RANK / K, 1e-6), 0.5)
    thresh = _stats.NormalDist().inv_cdf(1.0 - frac) / math.sqrt(D)

    qsel_raw, sel_gtsim, sel_gtidx, sel_valid, sel_flag = _sc_select(
        gtsim.reshape(B), qn, gt_idx, thresh)

    blk = _pick_block(K)
    nsteps = pl.cdiv(K, blk)
    body = functools.partial(_count_body, nsteps, blk, B, D, K, S)
    out = pl.pallas_call(
        body,
        grid=(nsteps,),
        compiler_params=pltpu.CompilerParams(
            vmem_limit_bytes=63 * 1024 * 1024),
        in_specs=[
            pl.BlockSpec((B, 1), lambda k: (0, 0)),
            pl.BlockSpec((B, 1), lambda k: (0, 0)),
            pl.BlockSpec((B, D), lambda k: (0, 0)),
            pl.BlockSpec((blk, D), lambda k: (k, 0)),
            pl.BlockSpec((S, D), lambda k: (0, 0)),
            pl.BlockSpec((1, S), lambda k: (0, 0)),
            pl.BlockSpec((1, S), lambda k: (0, 0)),
            pl.BlockSpec((1, S), lambda k: (0, 0)),
        ],
        out_specs=pl.BlockSpec((1, 1), lambda k: (0, 0)),
        out_shape=jax.ShapeDtypeStruct((1, 1), jnp.float32),
        scratch_shapes=[
            pltpu.VMEM((S, D), jnp.bfloat16),
            pltpu.VMEM((S, D), jnp.bfloat16),
            pltpu.VMEM((B, 1), jnp.int32),
            pltpu.VMEM((1, S), jnp.int32),
        ],
    )(gtsim, sel_flag.reshape(B, 1), qn16, keys,
      qsel_raw, sel_gtsim.reshape(1, S), sel_gtidx.reshape(1, S),
      sel_valid.reshape(1, S))
    return out.reshape(())
